# Initial kernel scaffold; baseline (speedup 1.0000x reference)
import jax
import jax.numpy as jnp
from jax.experimental import pallas as pl


def kernel(x, neigh, eps, W1, b1, W2, b2):
    raise NotImplementedError("write your pallas kernel here")


# speedup vs baseline: 5.4832x; 5.4832x over previous
import functools

import jax
import jax.numpy as jnp
from jax import lax
from jax.experimental import pallas as pl
from jax.experimental.pallas import tpu as pltpu
from jax.experimental.pallas import tpu_sc as plsc

N = 10000
K = 16
D = 256
LANES = 16
NC = 2
NS = 16
NW = NC * NS
NPW = 320
NP = NW * NPW
C = 4
CK = C * K
CHUNKS = NPW // C


def _agg_body(x_hbm, idx_hbm, out_hbm, idx_v, rows_v, agg_v, gsem):
    wid = lax.axis_index("s") * NC + lax.axis_index("c")
    pltpu.sync_copy(idx_hbm.at[wid], idx_v)
    pltpu.async_copy(x_hbm.at[idx_v.at[0]], rows_v.at[0], gsem)
    pltpu.async_copy(x_hbm.at[idx_v.at[1]], rows_v.at[1], gsem)

    def compute_chunk(c, b):
        def node_body(j, _):
            row0 = j * K

            def d_body(t, _):
                col = t * LANES
                s = rows_v[b, row0, pl.ds(col, LANES)]
                for k in range(1, K):
                    s = s + rows_v[b, row0 + k, pl.ds(col, LANES)]
                agg_v[c * C + j, pl.ds(col, LANES)] = s
                return 0

            return lax.fori_loop(0, D // LANES, d_body, 0)

        lax.fori_loop(0, C, node_body, 0)

    def pair_body(i, _):
        c0 = i * 2
        for b in range(2):
            c = c0 + b
            pltpu.make_async_copy(x_hbm.at[idx_v.at[c]], rows_v.at[b], gsem).wait()
            compute_chunk(c, b)
            pltpu.async_copy(x_hbm.at[idx_v.at[c + 2]], rows_v.at[b], gsem)
        return 0

    lax.fori_loop(0, (CHUNKS - 2) // 2, pair_body, 0)
    for b in range(2):
        c = CHUNKS - 2 + b
        pltpu.make_async_copy(x_hbm.at[idx_v.at[c]], rows_v.at[b], gsem).wait()
        compute_chunk(c, b)
    pltpu.sync_copy(agg_v, out_hbm.at[pl.ds(wid * NPW, NPW)])


@functools.cache
def _agg_call():
    mesh = plsc.VectorSubcoreMesh(core_axis_name="c", subcore_axis_name="s")
    return pl.kernel(
        _agg_body,
        out_type=jax.ShapeDtypeStruct((NP, D), jnp.float32),
        mesh=mesh,
        scratch_types=[
            pltpu.VMEM((CHUNKS, CK), jnp.int32),
            pltpu.VMEM((2, CK, D), jnp.float32),
            pltpu.VMEM((NPW, D), jnp.float32),
            pltpu.SemaphoreType.DMA,
        ],
    )


RT = 1000


def _mlp_body(eps_ref, x_ref, agg_ref, w1_ref, b1_ref, w2_ref, b2_ref, o_ref):
    h = (1.0 + eps_ref[0]) * x_ref[...] + agg_ref[...]
    h1 = jnp.dot(h, w1_ref[...], preferred_element_type=jnp.float32) + b1_ref[...]
    h1 = jnp.maximum(h1, 0.0)
    o_ref[...] = jnp.dot(h1, w2_ref[...], preferred_element_type=jnp.float32) + b2_ref[...]


@functools.cache
def _mlp_call():
    return pl.pallas_call(
        _mlp_body,
        grid=(N // RT,),
        in_specs=[
            pl.BlockSpec(memory_space=pltpu.SMEM),
            pl.BlockSpec((RT, D), lambda i: (i, 0)),
            pl.BlockSpec((RT, D), lambda i: (i, 0)),
            pl.BlockSpec((D, D), lambda i: (0, 0)),
            pl.BlockSpec((1, D), lambda i: (0, 0)),
            pl.BlockSpec((D, D), lambda i: (0, 0)),
            pl.BlockSpec((1, D), lambda i: (0, 0)),
        ],
        out_specs=pl.BlockSpec((RT, D), lambda i: (i, 0)),
        out_shape=jax.ShapeDtypeStruct((N, D), jnp.float32),
    )


def kernel(x, neigh, eps, W1, b1, W2, b2):
    x2d = x[0]
    idx = neigh.astype(jnp.int32)
    idx = jnp.pad(idx, ((0, NP - N), (0, 0)))
    idx = idx.reshape(NW, CHUNKS, CK)
    agg = _agg_call()(x2d, idx)
    eps_arr = jnp.reshape(eps, (1,)).astype(jnp.float32)
    out = _mlp_call()(eps_arr, x2d, agg, W1, jnp.reshape(b1, (1, D)),
                      W2, jnp.reshape(b2, (1, D)))
    return out[None]
